# baseline (speedup 1.0000x reference)
import functools

import jax
import jax.numpy as jnp
from jax import lax
from jax.experimental import pallas as pl
from jax.experimental.pallas import tpu as pltpu
from jax.experimental.pallas import tpu_sc as plsc

F32 = jnp.float32
BF16 = jnp.bfloat16

_NC = 2
_NS = 16
_NW = _NC * _NS


def _elu(y):
    return jnp.where(y > 0.0, y, jnp.exp(jnp.minimum(y, 0.0)) - 1.0)


def _proj_body(nf_ref, w_ref, pr_ref, q_ref, s_ref):
    c = nf_ref.shape[1]
    t = jnp.dot(nf_ref[...], w_ref[...], preferred_element_type=F32)
    pr_ref[...] = t[:, : 2 * c].astype(BF16)
    q_ref[...] = t[:, 2 * c : 3 * c].astype(BF16)
    s_ref[...] = t[:, 3 * c : 4 * c]


def _proj(nf, wcat, bn):
    n, c = nf.shape
    grid = (n // bn,)
    return pl.pallas_call(
        _proj_body,
        grid=grid,
        in_specs=[
            pl.BlockSpec((bn, c), lambda i: (i, 0)),
            pl.BlockSpec((c, 4 * c), lambda i: (0, 0)),
        ],
        out_specs=[
            pl.BlockSpec((bn, 2 * c), lambda i: (i, 0)),
            pl.BlockSpec((bn, c), lambda i: (i, 0)),
            pl.BlockSpec((bn, c), lambda i: (i, 0)),
        ],
        out_shape=[
            jax.ShapeDtypeStruct((n, 2 * c), BF16),
            jax.ShapeDtypeStruct((n, c), BF16),
            jax.ShapeDtypeStruct((n, c), F32),
        ],
    )(nf, wcat)


def _gather_call(pr, q, src, dst, e, c, k):
    ept = e // _NW
    nchunk = ept // k
    tail = ept - nchunk * k
    mesh = plsc.VectorSubcoreMesh(core_axis_name="c", subcore_axis_name="s")

    @functools.partial(
        pl.kernel,
        out_type=[
            jax.ShapeDtypeStruct((e, 2 * c), BF16),
            jax.ShapeDtypeStruct((e, c), BF16),
        ],
        mesh=mesh,
        scratch_types=[
            pltpu.VMEM((k,), jnp.int32),
            pltpu.VMEM((k,), jnp.int32),
            pltpu.VMEM((k, 2 * c), BF16),
            pltpu.VMEM((k, c), BF16),
            pltpu.SemaphoreType.DMA,
        ],
        compiler_params=pltpu.CompilerParams(use_tc_tiling_on_sc=False),
    )
    def kern(pr_h, q_h, src_h, dst_h, gpr_h, gq_h, srcv, dstv, prbuf, qbuf, sem):
        wid = lax.axis_index("s") * _NC + lax.axis_index("c")
        base = wid * ept

        def do_block(cb, nrow):
            pltpu.sync_copy(src_h.at[pl.ds(cb, nrow)], srcv.at[pl.ds(0, nrow)])
            pltpu.sync_copy(dst_h.at[pl.ds(cb, nrow)], dstv.at[pl.ds(0, nrow)])
            cps = []
            for o in range(0, nrow, 128):
                w = min(128, nrow - o)
                cps.append(pltpu.async_copy(
                    pr_h.at[srcv.at[pl.ds(o, w)]],
                    prbuf.at[pl.ds(o, w), :], sem))
                cps.append(pltpu.async_copy(
                    q_h.at[dstv.at[pl.ds(o, w)]],
                    qbuf.at[pl.ds(o, w), :], sem))
            for cp in cps:
                cp.wait()
            pltpu.sync_copy(prbuf.at[pl.ds(0, nrow), :],
                            gpr_h.at[pl.ds(cb, nrow), :])
            pltpu.sync_copy(qbuf.at[pl.ds(0, nrow), :],
                            gq_h.at[pl.ds(cb, nrow), :])

        def chunk(ci, carry):
            do_block(base + ci * k, k)
            return carry

        lax.fori_loop(0, nchunk, chunk, 0)
        if tail:
            do_block(base + nchunk * k, tail)

    return kern(pr, q, src, dst)


def _edge1_body(ef_ref, gp_ref, gq_ref, w_ref, z1_ref, st_ref):
    z = jnp.dot(ef_ref[...], w_ref[...], preferred_element_type=F32)
    z = z + gp_ref[...].astype(F32) + gq_ref[...].astype(F32)
    z1_ref[...] = z

    @pl.when(pl.program_id(0) == 0)
    def _():
        st_ref[...] = jnp.zeros_like(st_ref)

    s = jnp.sum(z, axis=0, keepdims=True)
    s2 = jnp.sum(z * z, axis=0, keepdims=True)
    st_ref[...] += jnp.concatenate([s, s2], axis=0)


def _edge1(ef, gpr, gq, w1c, be):
    e, c = ef.shape
    grid = (e // be,)
    return pl.pallas_call(
        _edge1_body,
        grid=grid,
        in_specs=[
            pl.BlockSpec((be, c), lambda i: (i, 0)),
            pl.BlockSpec((be, c), lambda i: (i, 0)),
            pl.BlockSpec((be, c), lambda i: (i, 0)),
            pl.BlockSpec((c, c), lambda i: (0, 0)),
        ],
        out_specs=[
            pl.BlockSpec((be, c), lambda i: (i, 0)),
            pl.BlockSpec((2, c), lambda i: (0, 0)),
        ],
        out_shape=[
            jax.ShapeDtypeStruct((e, c), F32),
            jax.ShapeDtypeStruct((2, c), F32),
        ],
    )(ef, gpr, gq, w1c)


def _edge2_body(nsteps, e, z1_ref, gr_ref, w_ref, st1_ref, g1_ref, be1_ref,
                g2_ref, be2_ref, ex_ref, z2_ref, af2_ref, acc_ref):
    inv_e = 1.0 / e
    mu = st1_ref[0:1, :] * inv_e
    var = jnp.maximum(st1_ref[1:2, :] * inv_e - mu * mu, 0.0)
    a1 = g1_ref[...] * lax.rsqrt(var + 1e-5)
    c1 = be1_ref[...] - mu * a1
    ex = _elu(z1_ref[...] * a1 + c1)
    ex_ref[...] = ex
    z2 = (jnp.dot(ex, w_ref[...], preferred_element_type=F32)
          + gr_ref[...].astype(F32))
    z2_ref[...] = z2

    @pl.when(pl.program_id(0) == 0)
    def _():
        acc_ref[...] = jnp.zeros_like(acc_ref)

    s = jnp.sum(z2, axis=0, keepdims=True)
    s2 = jnp.sum(z2 * z2, axis=0, keepdims=True)
    acc_ref[...] += jnp.concatenate([s, s2], axis=0)

    @pl.when(pl.program_id(0) == nsteps - 1)
    def _():
        mu2 = acc_ref[0:1, :] * inv_e
        var2 = jnp.maximum(acc_ref[1:2, :] * inv_e - mu2 * mu2, 0.0)
        a2 = g2_ref[...] * lax.rsqrt(var2 + 1e-5)
        c2 = be2_ref[...] - mu2 * a2
        af2_ref[...] = jnp.concatenate([a2, c2], axis=0)


def _edge2(z1, gpr, w2b, st1, g1, be1, g2, be2, be):
    e, c = z1.shape
    nsteps = e // be
    return pl.pallas_call(
        functools.partial(_edge2_body, nsteps, e),
        grid=(nsteps,),
        in_specs=[
            pl.BlockSpec((be, c), lambda i: (i, 0)),
            pl.BlockSpec((be, c), lambda i: (i, 1)),
            pl.BlockSpec((c, c), lambda i: (0, 0)),
            pl.BlockSpec((2, c), lambda i: (0, 0)),
            pl.BlockSpec((1, c), lambda i: (0, 0)),
            pl.BlockSpec((1, c), lambda i: (0, 0)),
            pl.BlockSpec((1, c), lambda i: (0, 0)),
            pl.BlockSpec((1, c), lambda i: (0, 0)),
        ],
        out_specs=[
            pl.BlockSpec((be, c), lambda i: (i, 0)),
            pl.BlockSpec((be, c), lambda i: (i, 0)),
            pl.BlockSpec((2, c), lambda i: (0, 0)),
        ],
        out_shape=[
            jax.ShapeDtypeStruct((e, c), F32),
            jax.ShapeDtypeStruct((e, c), F32),
            jax.ShapeDtypeStruct((2, c), F32),
        ],
        scratch_shapes=[pltpu.VMEM((2, c), F32)],
    )(z1, gpr, w2b, st1, g1, be1, g2, be2)


def _mpass_body(z2_ref, af2_ref, m_ref):
    be = z2_ref.shape[0]
    m = _elu(z2_ref[...] * af2_ref[0:1, :] + af2_ref[1:2, :])
    m_ref[...] = jnp.concatenate([m, jnp.ones((be, 16), F32)], axis=1)


def _mpass(z2, af2, be):
    e, c = z2.shape
    return pl.pallas_call(
        _mpass_body,
        grid=(e // be,),
        in_specs=[
            pl.BlockSpec((be, c), lambda i: (i, 0)),
            pl.BlockSpec((2, c), lambda i: (0, 0)),
        ],
        out_specs=pl.BlockSpec((be, c + 16), lambda i: (i, 0)),
        out_shape=jax.ShapeDtypeStruct((e, c + 16), F32),
    )(z2, af2)


def _scatter_call(m144, dst, n, e, c, k):
    ept = e // _NS
    nchunk = ept // k
    tail = ept - nchunk * k
    half = -(-n // _NC)
    rpt = (-(-(half + 1) // _NS) + 7) // 8 * 8
    nh_pad = rpt * _NS
    cw = c + 16
    ki = k // 128
    mesh = plsc.VectorSubcoreMesh(core_axis_name="c", subcore_axis_name="s")

    @functools.partial(
        pl.kernel,
        out_type=jax.ShapeDtypeStruct((_NC, nh_pad, cw), F32),
        mesh=mesh,
        scratch_types=[
            pltpu.VMEM((k, cw), F32),
            pltpu.VMEM((k,), jnp.int32),
            pltpu.VMEM((ki, 128), jnp.int32),
            pltpu.VMEM((rpt, cw), F32),
            pltpu.VMEM_SHARED((nh_pad, cw), F32),
        ],
        compiler_params=pltpu.CompilerParams(use_tc_tiling_on_sc=False),
    )
    def kern(m_h, dst_h, out_h, mbuf, dstv, idxb, dbuf, acc):
        cid = lax.axis_index("c")
        sid = lax.axis_index("s")
        base = sid * ept
        r0 = sid * rpt
        off = cid * half

        def zrow(i, carry):
            for j in range(cw // 16):
                dbuf[i, pl.ds(j * 16, 16)] = jnp.zeros((16,), F32)
            return carry

        lax.fori_loop(0, rpt, zrow, 0)
        pltpu.sync_copy(dbuf, acc.at[pl.ds(r0, rpt)])
        plsc.subcore_barrier()

        def do_block(cb, nrow):
            pltpu.sync_copy(m_h.at[pl.ds(cb, nrow), :],
                            mbuf.at[pl.ds(0, nrow), :])
            pltpu.sync_copy(dst_h.at[pl.ds(cb, nrow)], dstv.at[pl.ds(0, nrow)])
            prow = -(-nrow // 128) * 128
            for t in range(nrow // 16):
                sl = pl.ds(t * 16, 16)
                d = dstv[sl] - off
                ok = (d >= 0) & (d < half)
                idxb[t // 8, pl.ds((t % 8) * 16, 16)] = jnp.where(ok, d, half)
            for t in range(nrow // 16, prow // 16):
                idxb[t // 8, pl.ds((t % 8) * 16, 16)] = jnp.full(
                    (16,), half, jnp.int32)
            for t in range(0, prow, 128):
                pltpu.sync_copy(mbuf.at[pl.ds(t, 128), :],
                                acc.at[idxb.at[t // 128]], add=True)

        def chunk(ci, carry):
            do_block(base + ci * k, k)
            return carry

        lax.fori_loop(0, nchunk, chunk, 0)
        if tail:
            do_block(base + nchunk * k, tail)
        plsc.subcore_barrier()

        pltpu.sync_copy(acc.at[pl.ds(r0, rpt)], dbuf)
        pltpu.sync_copy(dbuf, out_h.at[cid, pl.ds(r0, rpt), :])

    return kern(m144, dst)


def _node_body(m0_ref, c0_ref, s_ref, w_ref, g3_ref, be3_ref, ho_ref):
    n = m0_ref.shape[0]
    msum = m0_ref[...]
    cnt = c0_ref[...][:, 0:1]
    hm = msum / jnp.maximum(cnt, 1.0)
    z3 = s_ref[...] + jnp.dot(hm, w_ref[...], preferred_element_type=F32)
    mu = jnp.sum(z3, axis=0, keepdims=True) * (1.0 / n)
    var = jnp.maximum(jnp.sum(z3 * z3, axis=0, keepdims=True) * (1.0 / n)
                      - mu * mu, 0.0)
    a3 = g3_ref[...] * lax.rsqrt(var + 1e-5)
    c3 = be3_ref[...] - mu * a3
    ho_ref[...] = _elu(z3 * a3 + c3)


def _node(m0, c0, s, w3b, g3, be3):
    n, c = s.shape
    return pl.pallas_call(
        _node_body,
        in_specs=[pl.BlockSpec(x.shape, lambda: tuple(0 for _ in x.shape))
                  for x in (m0, c0, s, w3b, g3, be3)],
        out_specs=pl.BlockSpec((n, c), lambda: (0, 0)),
        out_shape=jax.ShapeDtypeStruct((n, c), F32),
    )(m0, c0, s, w3b, g3, be3)


def kernel(node_feats, edge_feats, edge_index, W1, b1, g1, be1,
           W2, b2, g2, be2, W3, b3, g3, be3):
    n, c = node_feats.shape
    e = edge_feats.shape[0]
    src = edge_index[0]
    dst = edge_index[1]

    wcat = jnp.concatenate(
        [W1[:c], W2[:c], W1[c : 2 * c], W3[:c]], axis=1)
    w1c = W1[2 * c : 3 * c]
    w2b = W2[c : 2 * c]
    w3b = W3[c : 2 * c]
    g1r, be1r, g2r, be2r, g3r, be3r = (
        x.reshape(1, c) for x in (g1, be1, g2, be2, g3, be3))

    pr, q, s = _proj(node_feats, wcat, bn=2000)
    gpr, gq = _gather_call(pr, q, src, dst, e, c, k=256)
    z1, st1 = _edge1(edge_feats, gpr, gq, w1c, be=2560)
    ex, z2, af2 = _edge2(z1, gpr, w2b, st1, g1r, be1r, g2r, be2r, be=2560)
    m144 = _mpass(z2, af2, be=2560)
    part = _scatter_call(m144, dst, n, e, c, k=128)
    half = -(-n // _NC)
    msum = jnp.concatenate([part[0, :half, :c], part[1, : n - half, :c]])
    cnt16 = jnp.concatenate(
        [part[0, :half, c : c + 16], part[1, : n - half, c : c + 16]])
    ho = _node(msum, cnt16, s, w3b, g3r, be3r)
    return (ho, ex)


# speedup vs baseline: 1.2317x; 1.2317x over previous
import functools

import jax
import jax.numpy as jnp
from jax import lax
from jax.experimental import pallas as pl
from jax.experimental.pallas import tpu as pltpu
from jax.experimental.pallas import tpu_sc as plsc

F32 = jnp.float32

_NC = 2
_NS = 16
_NW = _NC * _NS


def _elu(y):
    return jnp.where(y > 0.0, y, jnp.exp(jnp.minimum(y, 0.0)) - 1.0)


def _proj_body(nf_ref, w_ref, pr_ref, q_ref, s_ref):
    c = nf_ref.shape[1]
    t = jnp.dot(nf_ref[...], w_ref[...], preferred_element_type=F32)
    pr_ref[...] = t[:, : 2 * c]
    q_ref[...] = t[:, 2 * c : 3 * c]
    s_ref[...] = t[:, 3 * c : 4 * c]


def _proj(nf, wcat, bn):
    n, c = nf.shape
    grid = (n // bn,)
    return pl.pallas_call(
        _proj_body,
        grid=grid,
        in_specs=[
            pl.BlockSpec((bn, c), lambda i: (i, 0)),
            pl.BlockSpec((c, 4 * c), lambda i: (0, 0)),
        ],
        out_specs=[
            pl.BlockSpec((bn, 2 * c), lambda i: (i, 0)),
            pl.BlockSpec((bn, c), lambda i: (i, 0)),
            pl.BlockSpec((bn, c), lambda i: (i, 0)),
        ],
        out_shape=[
            jax.ShapeDtypeStruct((n, 2 * c), F32),
            jax.ShapeDtypeStruct((n, c), F32),
            jax.ShapeDtypeStruct((n, c), F32),
        ],
    )(nf, wcat)


def _gather_call(pr, q, src, dst, e, c, k):
    ept = e // _NW
    nchunk = ept // k
    tail = ept - nchunk * k
    mesh = plsc.VectorSubcoreMesh(core_axis_name="c", subcore_axis_name="s")

    @functools.partial(
        pl.kernel,
        out_type=[
            jax.ShapeDtypeStruct((e, 2 * c), F32),
            jax.ShapeDtypeStruct((e, c), F32),
        ],
        mesh=mesh,
        scratch_types=[
            pltpu.VMEM((k,), jnp.int32),
            pltpu.VMEM((k,), jnp.int32),
            pltpu.VMEM((k, 2 * c), F32),
            pltpu.VMEM((k, c), F32),
            pltpu.SemaphoreType.DMA,
        ],
        compiler_params=pltpu.CompilerParams(use_tc_tiling_on_sc=False),
    )
    def kern(pr_h, q_h, src_h, dst_h, gpr_h, gq_h, srcv, dstv, prbuf, qbuf, sem):
        wid = lax.axis_index("s") * _NC + lax.axis_index("c")
        base = wid * ept

        def do_block(cb, nrow):
            pltpu.sync_copy(src_h.at[pl.ds(cb, nrow)], srcv.at[pl.ds(0, nrow)])
            pltpu.sync_copy(dst_h.at[pl.ds(cb, nrow)], dstv.at[pl.ds(0, nrow)])
            cps = []
            for o in range(0, nrow, 128):
                w = min(128, nrow - o)
                cps.append(pltpu.async_copy(
                    pr_h.at[srcv.at[pl.ds(o, w)]],
                    prbuf.at[pl.ds(o, w), :], sem))
                cps.append(pltpu.async_copy(
                    q_h.at[dstv.at[pl.ds(o, w)]],
                    qbuf.at[pl.ds(o, w), :], sem))
            for cp in cps:
                cp.wait()
            pltpu.sync_copy(prbuf.at[pl.ds(0, nrow), :],
                            gpr_h.at[pl.ds(cb, nrow), :])
            pltpu.sync_copy(qbuf.at[pl.ds(0, nrow), :],
                            gq_h.at[pl.ds(cb, nrow), :])

        def chunk(ci, carry):
            do_block(base + ci * k, k)
            return carry

        lax.fori_loop(0, nchunk, chunk, 0)
        if tail:
            do_block(base + nchunk * k, tail)

    return kern(pr, q, src, dst)


def _edge1_body(ef_ref, gp_ref, gq_ref, w_ref, z1_ref, st_ref):
    z = jnp.dot(ef_ref[...], w_ref[...], preferred_element_type=F32)
    z = z + gp_ref[...] + gq_ref[...]
    z1_ref[...] = z

    @pl.when(pl.program_id(0) == 0)
    def _():
        st_ref[...] = jnp.zeros_like(st_ref)

    s = jnp.sum(z, axis=0, keepdims=True)
    s2 = jnp.sum(z * z, axis=0, keepdims=True)
    st_ref[...] += jnp.concatenate([s, s2], axis=0)


def _edge1(ef, gpr, gq, w1c, be):
    e, c = ef.shape
    grid = (e // be,)
    return pl.pallas_call(
        _edge1_body,
        grid=grid,
        in_specs=[
            pl.BlockSpec((be, c), lambda i: (i, 0)),
            pl.BlockSpec((be, c), lambda i: (i, 0)),
            pl.BlockSpec((be, c), lambda i: (i, 0)),
            pl.BlockSpec((c, c), lambda i: (0, 0)),
        ],
        out_specs=[
            pl.BlockSpec((be, c), lambda i: (i, 0)),
            pl.BlockSpec((2, c), lambda i: (0, 0)),
        ],
        out_shape=[
            jax.ShapeDtypeStruct((e, c), F32),
            jax.ShapeDtypeStruct((2, c), F32),
        ],
    )(ef, gpr, gq, w1c)


def _edge2_body(nsteps, e, z1_ref, gr_ref, w_ref, st1_ref, g1_ref, be1_ref,
                g2_ref, be2_ref, ex_ref, z2_ref, af2_ref, acc_ref):
    inv_e = 1.0 / e
    mu = st1_ref[0:1, :] * inv_e
    var = jnp.maximum(st1_ref[1:2, :] * inv_e - mu * mu, 0.0)
    a1 = g1_ref[...] * lax.rsqrt(var + 1e-5)
    c1 = be1_ref[...] - mu * a1
    ex = _elu(z1_ref[...] * a1 + c1)
    ex_ref[...] = ex
    z2 = jnp.dot(ex, w_ref[...], preferred_element_type=F32) + gr_ref[...]
    z2_ref[...] = z2

    @pl.when(pl.program_id(0) == 0)
    def _():
        acc_ref[...] = jnp.zeros_like(acc_ref)

    s = jnp.sum(z2, axis=0, keepdims=True)
    s2 = jnp.sum(z2 * z2, axis=0, keepdims=True)
    acc_ref[...] += jnp.concatenate([s, s2], axis=0)

    @pl.when(pl.program_id(0) == nsteps - 1)
    def _():
        mu2 = acc_ref[0:1, :] * inv_e
        var2 = jnp.maximum(acc_ref[1:2, :] * inv_e - mu2 * mu2, 0.0)
        a2 = g2_ref[...] * lax.rsqrt(var2 + 1e-5)
        c2 = be2_ref[...] - mu2 * a2
        af2_ref[...] = jnp.concatenate([a2, c2], axis=0)


def _edge2(z1, gpr, w2b, st1, g1, be1, g2, be2, be):
    e, c = z1.shape
    nsteps = e // be
    return pl.pallas_call(
        functools.partial(_edge2_body, nsteps, e),
        grid=(nsteps,),
        in_specs=[
            pl.BlockSpec((be, c), lambda i: (i, 0)),
            pl.BlockSpec((be, c), lambda i: (i, 1)),
            pl.BlockSpec((c, c), lambda i: (0, 0)),
            pl.BlockSpec((2, c), lambda i: (0, 0)),
            pl.BlockSpec((1, c), lambda i: (0, 0)),
            pl.BlockSpec((1, c), lambda i: (0, 0)),
            pl.BlockSpec((1, c), lambda i: (0, 0)),
            pl.BlockSpec((1, c), lambda i: (0, 0)),
        ],
        out_specs=[
            pl.BlockSpec((be, c), lambda i: (i, 0)),
            pl.BlockSpec((be, c), lambda i: (i, 0)),
            pl.BlockSpec((2, c), lambda i: (0, 0)),
        ],
        out_shape=[
            jax.ShapeDtypeStruct((e, c), F32),
            jax.ShapeDtypeStruct((e, c), F32),
            jax.ShapeDtypeStruct((2, c), F32),
        ],
        scratch_shapes=[pltpu.VMEM((2, c), F32)],
    )(z1, gpr, w2b, st1, g1, be1, g2, be2)


def _mpass_body(z2_ref, af2_ref, m_ref):
    be = z2_ref.shape[0]
    m = _elu(z2_ref[...] * af2_ref[0:1, :] + af2_ref[1:2, :])
    m_ref[...] = jnp.concatenate([m, jnp.ones((be, 16), F32)], axis=1)


def _mpass(z2, af2, be):
    e, c = z2.shape
    return pl.pallas_call(
        _mpass_body,
        grid=(e // be,),
        in_specs=[
            pl.BlockSpec((be, c), lambda i: (i, 0)),
            pl.BlockSpec((2, c), lambda i: (0, 0)),
        ],
        out_specs=pl.BlockSpec((be, c + 16), lambda i: (i, 0)),
        out_shape=jax.ShapeDtypeStruct((e, c + 16), F32),
    )(z2, af2)


def _scatter_call(m144, dst, n, e, c, k):
    ept = e // _NS
    nchunk = ept // k
    tail = ept - nchunk * k
    half = -(-n // _NC)
    rpt = (-(-(half + 1) // _NS) + 7) // 8 * 8
    nh_pad = rpt * _NS
    cw = c + 16
    ki = k // 128
    mesh = plsc.VectorSubcoreMesh(core_axis_name="c", subcore_axis_name="s")

    @functools.partial(
        pl.kernel,
        out_type=jax.ShapeDtypeStruct((_NC, nh_pad, cw), F32),
        mesh=mesh,
        scratch_types=[
            pltpu.VMEM((k, cw), F32),
            pltpu.VMEM((k, cw), F32),
            pltpu.VMEM((k,), jnp.int32),
            pltpu.VMEM((k,), jnp.int32),
            pltpu.VMEM((ki, 128), jnp.int32),
            pltpu.VMEM((rpt, cw), F32),
            pltpu.VMEM_SHARED((nh_pad, cw), F32),
            pltpu.SemaphoreType.DMA,
            pltpu.SemaphoreType.DMA,
        ],
        compiler_params=pltpu.CompilerParams(use_tc_tiling_on_sc=False),
    )
    def kern(m_h, dst_h, out_h, mb0, mb1, dv0, dv1, idxb, dbuf, acc, sm0, sm1):
        cid = lax.axis_index("c")
        sid = lax.axis_index("s")
        base = sid * ept
        r0 = sid * rpt
        off = cid * half
        mbufs, dvs, sems = (mb0, mb1), (dv0, dv1), (sm0, sm1)

        def zrow(i, carry):
            for j in range(cw // 16):
                dbuf[i, pl.ds(j * 16, 16)] = jnp.zeros((16,), F32)
            return carry

        lax.fori_loop(0, rpt, zrow, 0)
        pltpu.sync_copy(dbuf, acc.at[pl.ds(r0, rpt)])
        plsc.subcore_barrier()

        def start_load(ci, b):
            cb = base + ci * k
            pltpu.async_copy(m_h.at[pl.ds(cb, k), :], mbufs[b], sems[b])
            pltpu.async_copy(dst_h.at[pl.ds(cb, k)], dvs[b], sems[b])

        def drain_load(b):
            pltpu.make_async_copy(m_h.at[pl.ds(0, k), :], mbufs[b],
                                  sems[b]).wait()
            pltpu.make_async_copy(dst_h.at[pl.ds(0, k)], dvs[b],
                                  sems[b]).wait()

        def process(b):
            for t in range(k // 16):
                sl = pl.ds(t * 16, 16)
                d = dvs[b][sl] - off
                ok = (d >= 0) & (d < half)
                idxb[t // 8, pl.ds((t % 8) * 16, 16)] = jnp.where(ok, d, half)
            for t in range(0, k, 128):
                pltpu.sync_copy(mbufs[b].at[pl.ds(t, 128), :],
                                acc.at[idxb.at[t // 128]], add=True)

        start_load(0, 0)

        def chunk2(c2, carry):
            ci = c2 * 2
            start_load(ci + 1, 1)
            drain_load(0)
            process(0)
            nxt = jnp.minimum(ci + 2, nchunk - 1)
            start_load(nxt, 0)
            drain_load(1)
            process(1)
            return carry

        lax.fori_loop(0, nchunk // 2, chunk2, 0)
        drain_load(0)

        if tail:
            cb = base + nchunk * k
            nrow = tail
            pltpu.sync_copy(m_h.at[pl.ds(cb, nrow), :],
                            mb0.at[pl.ds(0, nrow), :])
            pltpu.sync_copy(dst_h.at[pl.ds(cb, nrow)], dv0.at[pl.ds(0, nrow)])
            prow = -(-nrow // 128) * 128
            for t in range(nrow // 16):
                sl = pl.ds(t * 16, 16)
                d = dv0[sl] - off
                ok = (d >= 0) & (d < half)
                idxb[t // 8, pl.ds((t % 8) * 16, 16)] = jnp.where(ok, d, half)
            for t in range(nrow // 16, prow // 16):
                idxb[t // 8, pl.ds((t % 8) * 16, 16)] = jnp.full(
                    (16,), half, jnp.int32)
            for t in range(0, prow, 128):
                pltpu.sync_copy(mb0.at[pl.ds(t, 128), :],
                                acc.at[idxb.at[t // 128]], add=True)
        plsc.subcore_barrier()

        pltpu.sync_copy(acc.at[pl.ds(r0, rpt)], dbuf)
        pltpu.sync_copy(dbuf, out_h.at[cid, pl.ds(r0, rpt), :])

    return kern(m144, dst)


def _node_body(m0_ref, c0_ref, s_ref, w_ref, g3_ref, be3_ref, ho_ref):
    n = m0_ref.shape[0]
    msum = m0_ref[...]
    cnt = c0_ref[...][:, 0:1]
    hm = msum / jnp.maximum(cnt, 1.0)
    z3 = s_ref[...] + jnp.dot(hm, w_ref[...], preferred_element_type=F32)
    mu = jnp.sum(z3, axis=0, keepdims=True) * (1.0 / n)
    var = jnp.maximum(jnp.sum(z3 * z3, axis=0, keepdims=True) * (1.0 / n)
                      - mu * mu, 0.0)
    a3 = g3_ref[...] * lax.rsqrt(var + 1e-5)
    c3 = be3_ref[...] - mu * a3
    ho_ref[...] = _elu(z3 * a3 + c3)


def _node(m0, c0, s, w3b, g3, be3):
    n, c = s.shape
    return pl.pallas_call(
        _node_body,
        in_specs=[pl.BlockSpec(x.shape, lambda: tuple(0 for _ in x.shape))
                  for x in (m0, c0, s, w3b, g3, be3)],
        out_specs=pl.BlockSpec((n, c), lambda: (0, 0)),
        out_shape=jax.ShapeDtypeStruct((n, c), F32),
    )(m0, c0, s, w3b, g3, be3)


def kernel(node_feats, edge_feats, edge_index, W1, b1, g1, be1,
           W2, b2, g2, be2, W3, b3, g3, be3):
    n, c = node_feats.shape
    e = edge_feats.shape[0]
    src = edge_index[0]
    dst = edge_index[1]

    wcat = jnp.concatenate(
        [W1[:c], W2[:c], W1[c : 2 * c], W3[:c]], axis=1)
    w1c = W1[2 * c : 3 * c]
    w2b = W2[c : 2 * c]
    w3b = W3[c : 2 * c]
    g1r, be1r, g2r, be2r, g3r, be3r = (
        x.reshape(1, c) for x in (g1, be1, g2, be2, g3, be3))

    pr, q, s = _proj(node_feats, wcat, bn=2000)
    gpr, gq = _gather_call(pr, q, src, dst, e, c, k=256)
    z1, st1 = _edge1(edge_feats, gpr, gq, w1c, be=4000)
    ex, z2, af2 = _edge2(z1, gpr, w2b, st1, g1r, be1r, g2r, be2r, be=4000)
    m144 = _mpass(z2, af2, be=4000)
    part = _scatter_call(m144, dst, n, e, c, k=128)
    half = -(-n // _NC)
    msum = jnp.concatenate([part[0, :half, :c], part[1, : n - half, :c]])
    cnt16 = jnp.concatenate(
        [part[0, :half, c : c + 16], part[1, : n - half, c : c + 16]])
    ho = _node(msum, cnt16, s, w3b, g3r, be3r)
    return (ho, ex)


# speedup vs baseline: 1.2361x; 1.0035x over previous
import functools

import jax
import jax.numpy as jnp
from jax import lax
from jax.experimental import pallas as pl
from jax.experimental.pallas import tpu as pltpu
from jax.experimental.pallas import tpu_sc as plsc

F32 = jnp.float32

_NC = 2
_NS = 16
_NW = _NC * _NS


def _elu(y):
    return jnp.where(y > 0.0, y, jnp.exp(jnp.minimum(y, 0.0)) - 1.0)


def _proj_body(nf_ref, w_ref, pr_ref, q_ref, s_ref):
    c = nf_ref.shape[1]
    t = jnp.dot(nf_ref[...], w_ref[...], preferred_element_type=F32)
    pr_ref[...] = t[:, : 2 * c]
    q_ref[...] = t[:, 2 * c : 3 * c]
    s_ref[...] = t[:, 3 * c : 4 * c]


def _proj(nf, wcat, bn):
    n, c = nf.shape
    grid = (n // bn,)
    return pl.pallas_call(
        _proj_body,
        grid=grid,
        in_specs=[
            pl.BlockSpec((bn, c), lambda i: (i, 0)),
            pl.BlockSpec((c, 4 * c), lambda i: (0, 0)),
        ],
        out_specs=[
            pl.BlockSpec((bn, 2 * c), lambda i: (i, 0)),
            pl.BlockSpec((bn, c), lambda i: (i, 0)),
            pl.BlockSpec((bn, c), lambda i: (i, 0)),
        ],
        out_shape=[
            jax.ShapeDtypeStruct((n, 2 * c), F32),
            jax.ShapeDtypeStruct((n, c), F32),
            jax.ShapeDtypeStruct((n, c), F32),
        ],
    )(nf, wcat)


def _gather_call(pr, q, src, dst, e, c, k):
    ept = e // _NW
    nchunk = ept // k
    tail = ept - nchunk * k
    mesh = plsc.VectorSubcoreMesh(core_axis_name="c", subcore_axis_name="s")

    @functools.partial(
        pl.kernel,
        out_type=[
            jax.ShapeDtypeStruct((e, 2 * c), F32),
            jax.ShapeDtypeStruct((e, c), F32),
        ],
        mesh=mesh,
        scratch_types=[
            pltpu.VMEM((k,), jnp.int32),
            pltpu.VMEM((k,), jnp.int32),
            pltpu.VMEM((k,), jnp.int32),
            pltpu.VMEM((k,), jnp.int32),
            pltpu.VMEM((k, 2 * c), F32),
            pltpu.VMEM((k, 2 * c), F32),
            pltpu.VMEM((k, c), F32),
            pltpu.VMEM((k, c), F32),
            pltpu.SemaphoreType.DMA,
            pltpu.SemaphoreType.DMA,
        ],
        compiler_params=pltpu.CompilerParams(use_tc_tiling_on_sc=False),
    )
    def kern(pr_h, q_h, src_h, dst_h, gpr_h, gq_h,
             sv0, sv1, dv0, dv1, pb0, pb1, qb0, qb1, sm0, sm1):
        wid = lax.axis_index("s") * _NC + lax.axis_index("c")
        base = wid * ept
        svs, dvs = (sv0, sv1), (dv0, dv1)
        pbs, qbs, sems = (pb0, pb1), (qb0, qb1), (sm0, sm1)

        def start_gather(ci, b):
            cb = base + ci * k
            pltpu.sync_copy(src_h.at[pl.ds(cb, k)], svs[b])
            pltpu.sync_copy(dst_h.at[pl.ds(cb, k)], dvs[b])
            pltpu.async_copy(pr_h.at[svs[b]], pbs[b], sems[b])
            pltpu.async_copy(q_h.at[dvs[b]], qbs[b], sems[b])

        def drain_gather(b):
            pltpu.make_async_copy(pr_h.at[svs[b]], pbs[b], sems[b]).wait()
            pltpu.make_async_copy(q_h.at[dvs[b]], qbs[b], sems[b]).wait()

        def write_out(ci, b):
            cb = base + ci * k
            pltpu.sync_copy(pbs[b], gpr_h.at[pl.ds(cb, k), :])
            pltpu.sync_copy(qbs[b], gq_h.at[pl.ds(cb, k), :])

        start_gather(0, 0)

        def chunk2(c2, carry):
            ci = c2 * 2
            start_gather(ci + 1, 1)
            drain_gather(0)
            write_out(ci, 0)
            nxt = jnp.minimum(ci + 2, nchunk - 1)
            start_gather(nxt, 0)
            drain_gather(1)
            write_out(ci + 1, 1)
            return carry

        lax.fori_loop(0, nchunk // 2, chunk2, 0)
        drain_gather(0)

        if tail:
            cb = base + nchunk * k
            pltpu.sync_copy(src_h.at[pl.ds(cb, tail)], sv0.at[pl.ds(0, tail)])
            pltpu.sync_copy(dst_h.at[pl.ds(cb, tail)], dv0.at[pl.ds(0, tail)])
            cp1 = pltpu.async_copy(pr_h.at[sv0.at[pl.ds(0, tail)]],
                                   pb0.at[pl.ds(0, tail), :], sm0)
            cp2 = pltpu.async_copy(q_h.at[dv0.at[pl.ds(0, tail)]],
                                   qb0.at[pl.ds(0, tail), :], sm0)
            cp1.wait()
            cp2.wait()
            pltpu.sync_copy(pb0.at[pl.ds(0, tail), :],
                            gpr_h.at[pl.ds(cb, tail), :])
            pltpu.sync_copy(qb0.at[pl.ds(0, tail), :],
                            gq_h.at[pl.ds(cb, tail), :])

    return kern(pr, q, src, dst)


def _edge1_body(ef_ref, gp_ref, gq_ref, w_ref, z1_ref, st_ref):
    z = jnp.dot(ef_ref[...], w_ref[...], preferred_element_type=F32)
    z = z + gp_ref[...] + gq_ref[...]
    z1_ref[...] = z

    @pl.when(pl.program_id(0) == 0)
    def _():
        st_ref[...] = jnp.zeros_like(st_ref)

    s = jnp.sum(z, axis=0, keepdims=True)
    s2 = jnp.sum(z * z, axis=0, keepdims=True)
    st_ref[...] += jnp.concatenate([s, s2], axis=0)


def _edge1(ef, gpr, gq, w1c, be):
    e, c = ef.shape
    grid = (e // be,)
    return pl.pallas_call(
        _edge1_body,
        grid=grid,
        in_specs=[
            pl.BlockSpec((be, c), lambda i: (i, 0)),
            pl.BlockSpec((be, c), lambda i: (i, 0)),
            pl.BlockSpec((be, c), lambda i: (i, 0)),
            pl.BlockSpec((c, c), lambda i: (0, 0)),
        ],
        out_specs=[
            pl.BlockSpec((be, c), lambda i: (i, 0)),
            pl.BlockSpec((2, c), lambda i: (0, 0)),
        ],
        out_shape=[
            jax.ShapeDtypeStruct((e, c), F32),
            jax.ShapeDtypeStruct((2, c), F32),
        ],
    )(ef, gpr, gq, w1c)


def _edge2_body(nsteps, e, z1_ref, gr_ref, w_ref, st1_ref, g1_ref, be1_ref,
                g2_ref, be2_ref, ex_ref, z2_ref, af2_ref, acc_ref):
    inv_e = 1.0 / e
    mu = st1_ref[0:1, :] * inv_e
    var = jnp.maximum(st1_ref[1:2, :] * inv_e - mu * mu, 0.0)
    a1 = g1_ref[...] * lax.rsqrt(var + 1e-5)
    c1 = be1_ref[...] - mu * a1
    ex = _elu(z1_ref[...] * a1 + c1)
    ex_ref[...] = ex
    z2 = jnp.dot(ex, w_ref[...], preferred_element_type=F32) + gr_ref[...]
    z2_ref[...] = z2

    @pl.when(pl.program_id(0) == 0)
    def _():
        acc_ref[...] = jnp.zeros_like(acc_ref)

    s = jnp.sum(z2, axis=0, keepdims=True)
    s2 = jnp.sum(z2 * z2, axis=0, keepdims=True)
    acc_ref[...] += jnp.concatenate([s, s2], axis=0)

    @pl.when(pl.program_id(0) == nsteps - 1)
    def _():
        mu2 = acc_ref[0:1, :] * inv_e
        var2 = jnp.maximum(acc_ref[1:2, :] * inv_e - mu2 * mu2, 0.0)
        a2 = g2_ref[...] * lax.rsqrt(var2 + 1e-5)
        c2 = be2_ref[...] - mu2 * a2
        af2_ref[...] = jnp.concatenate([a2, c2], axis=0)


def _edge2(z1, gpr, w2b, st1, g1, be1, g2, be2, be):
    e, c = z1.shape
    nsteps = e // be
    return pl.pallas_call(
        functools.partial(_edge2_body, nsteps, e),
        grid=(nsteps,),
        in_specs=[
            pl.BlockSpec((be, c), lambda i: (i, 0)),
            pl.BlockSpec((be, c), lambda i: (i, 1)),
            pl.BlockSpec((c, c), lambda i: (0, 0)),
            pl.BlockSpec((2, c), lambda i: (0, 0)),
            pl.BlockSpec((1, c), lambda i: (0, 0)),
            pl.BlockSpec((1, c), lambda i: (0, 0)),
            pl.BlockSpec((1, c), lambda i: (0, 0)),
            pl.BlockSpec((1, c), lambda i: (0, 0)),
        ],
        out_specs=[
            pl.BlockSpec((be, c), lambda i: (i, 0)),
            pl.BlockSpec((be, c), lambda i: (i, 0)),
            pl.BlockSpec((2, c), lambda i: (0, 0)),
        ],
        out_shape=[
            jax.ShapeDtypeStruct((e, c), F32),
            jax.ShapeDtypeStruct((e, c), F32),
            jax.ShapeDtypeStruct((2, c), F32),
        ],
        scratch_shapes=[pltpu.VMEM((2, c), F32)],
    )(z1, gpr, w2b, st1, g1, be1, g2, be2)


def _mpass_body(z2_ref, af2_ref, m_ref):
    be = z2_ref.shape[0]
    m = _elu(z2_ref[...] * af2_ref[0:1, :] + af2_ref[1:2, :])
    m_ref[...] = jnp.concatenate([m, jnp.ones((be, 16), F32)], axis=1)


def _mpass(z2, af2, be):
    e, c = z2.shape
    return pl.pallas_call(
        _mpass_body,
        grid=(e // be,),
        in_specs=[
            pl.BlockSpec((be, c), lambda i: (i, 0)),
            pl.BlockSpec((2, c), lambda i: (0, 0)),
        ],
        out_specs=pl.BlockSpec((be, c + 16), lambda i: (i, 0)),
        out_shape=jax.ShapeDtypeStruct((e, c + 16), F32),
    )(z2, af2)


def _scatter_call(m144, dst, n, e, c, k):
    ept = e // _NS
    nchunk = ept // k
    tail = ept - nchunk * k
    half = -(-n // _NC)
    rpt = (-(-(half + 1) // _NS) + 7) // 8 * 8
    nh_pad = rpt * _NS
    cw = c + 16
    ki = k // 128
    mesh = plsc.VectorSubcoreMesh(core_axis_name="c", subcore_axis_name="s")

    @functools.partial(
        pl.kernel,
        out_type=jax.ShapeDtypeStruct((_NC, nh_pad, cw), F32),
        mesh=mesh,
        scratch_types=[
            pltpu.VMEM((k, cw), F32),
            pltpu.VMEM((k, cw), F32),
            pltpu.VMEM((k,), jnp.int32),
            pltpu.VMEM((k,), jnp.int32),
            pltpu.VMEM((ki, 128), jnp.int32),
            pltpu.VMEM((rpt, cw), F32),
            pltpu.VMEM_SHARED((nh_pad, cw), F32),
            pltpu.SemaphoreType.DMA,
            pltpu.SemaphoreType.DMA,
        ],
        compiler_params=pltpu.CompilerParams(use_tc_tiling_on_sc=False),
    )
    def kern(m_h, dst_h, out_h, mb0, mb1, dv0, dv1, idxb, dbuf, acc, sm0, sm1):
        cid = lax.axis_index("c")
        sid = lax.axis_index("s")
        base = sid * ept
        r0 = sid * rpt
        off = cid * half
        mbufs, dvs, sems = (mb0, mb1), (dv0, dv1), (sm0, sm1)

        def zrow(i, carry):
            for j in range(cw // 16):
                dbuf[i, pl.ds(j * 16, 16)] = jnp.zeros((16,), F32)
            return carry

        lax.fori_loop(0, rpt, zrow, 0)
        pltpu.sync_copy(dbuf, acc.at[pl.ds(r0, rpt)])
        plsc.subcore_barrier()

        def start_load(ci, b):
            cb = base + ci * k
            pltpu.async_copy(m_h.at[pl.ds(cb, k), :], mbufs[b], sems[b])
            pltpu.async_copy(dst_h.at[pl.ds(cb, k)], dvs[b], sems[b])

        def drain_load(b):
            pltpu.make_async_copy(m_h.at[pl.ds(0, k), :], mbufs[b],
                                  sems[b]).wait()
            pltpu.make_async_copy(dst_h.at[pl.ds(0, k)], dvs[b],
                                  sems[b]).wait()

        def process(b):
            for t in range(k // 16):
                sl = pl.ds(t * 16, 16)
                d = dvs[b][sl] - off
                ok = (d >= 0) & (d < half)
                idxb[t // 8, pl.ds((t % 8) * 16, 16)] = jnp.where(ok, d, half)
            for t in range(0, k, 128):
                pltpu.sync_copy(mbufs[b].at[pl.ds(t, 128), :],
                                acc.at[idxb.at[t // 128]], add=True)

        start_load(0, 0)

        def chunk2(c2, carry):
            ci = c2 * 2
            start_load(ci + 1, 1)
            drain_load(0)
            process(0)
            nxt = jnp.minimum(ci + 2, nchunk - 1)
            start_load(nxt, 0)
            drain_load(1)
            process(1)
            return carry

        lax.fori_loop(0, nchunk // 2, chunk2, 0)
        drain_load(0)

        if tail:
            cb = base + nchunk * k
            nrow = tail
            pltpu.sync_copy(m_h.at[pl.ds(cb, nrow), :],
                            mb0.at[pl.ds(0, nrow), :])
            pltpu.sync_copy(dst_h.at[pl.ds(cb, nrow)], dv0.at[pl.ds(0, nrow)])
            prow = -(-nrow // 128) * 128
            for t in range(nrow // 16):
                sl = pl.ds(t * 16, 16)
                d = dv0[sl] - off
                ok = (d >= 0) & (d < half)
                idxb[t // 8, pl.ds((t % 8) * 16, 16)] = jnp.where(ok, d, half)
            for t in range(nrow // 16, prow // 16):
                idxb[t // 8, pl.ds((t % 8) * 16, 16)] = jnp.full(
                    (16,), half, jnp.int32)
            for t in range(0, prow, 128):
                pltpu.sync_copy(mb0.at[pl.ds(t, 128), :],
                                acc.at[idxb.at[t // 128]], add=True)
        plsc.subcore_barrier()

        pltpu.sync_copy(acc.at[pl.ds(r0, rpt)], dbuf)
        pltpu.sync_copy(dbuf, out_h.at[cid, pl.ds(r0, rpt), :])

    return kern(m144, dst)


def _node_body(m0_ref, c0_ref, s_ref, w_ref, g3_ref, be3_ref, ho_ref):
    n = m0_ref.shape[0]
    msum = m0_ref[...]
    cnt = c0_ref[...][:, 0:1]
    hm = msum / jnp.maximum(cnt, 1.0)
    z3 = s_ref[...] + jnp.dot(hm, w_ref[...], preferred_element_type=F32)
    mu = jnp.sum(z3, axis=0, keepdims=True) * (1.0 / n)
    var = jnp.maximum(jnp.sum(z3 * z3, axis=0, keepdims=True) * (1.0 / n)
                      - mu * mu, 0.0)
    a3 = g3_ref[...] * lax.rsqrt(var + 1e-5)
    c3 = be3_ref[...] - mu * a3
    ho_ref[...] = _elu(z3 * a3 + c3)


def _node(m0, c0, s, w3b, g3, be3):
    n, c = s.shape
    return pl.pallas_call(
        _node_body,
        in_specs=[pl.BlockSpec(x.shape, lambda: tuple(0 for _ in x.shape))
                  for x in (m0, c0, s, w3b, g3, be3)],
        out_specs=pl.BlockSpec((n, c), lambda: (0, 0)),
        out_shape=jax.ShapeDtypeStruct((n, c), F32),
    )(m0, c0, s, w3b, g3, be3)


def kernel(node_feats, edge_feats, edge_index, W1, b1, g1, be1,
           W2, b2, g2, be2, W3, b3, g3, be3):
    n, c = node_feats.shape
    e = edge_feats.shape[0]
    src = edge_index[0]
    dst = edge_index[1]

    wcat = jnp.concatenate(
        [W1[:c], W2[:c], W1[c : 2 * c], W3[:c]], axis=1)
    w1c = W1[2 * c : 3 * c]
    w2b = W2[c : 2 * c]
    w3b = W3[c : 2 * c]
    g1r, be1r, g2r, be2r, g3r, be3r = (
        x.reshape(1, c) for x in (g1, be1, g2, be2, g3, be3))

    pr, q, s = _proj(node_feats, wcat, bn=2000)
    gpr, gq = _gather_call(pr, q, src, dst, e, c, k=128)
    z1, st1 = _edge1(edge_feats, gpr, gq, w1c, be=4000)
    ex, z2, af2 = _edge2(z1, gpr, w2b, st1, g1r, be1r, g2r, be2r, be=4000)
    m144 = _mpass(z2, af2, be=4000)
    part = _scatter_call(m144, dst, n, e, c, k=128)
    half = -(-n // _NC)
    msum = jnp.concatenate([part[0, :half, :c], part[1, : n - half, :c]])
    cnt16 = jnp.concatenate(
        [part[0, :half, c : c + 16], part[1, : n - half, c : c + 16]])
    ho = _node(msum, cnt16, s, w3b, g3r, be3r)
    return (ho, ex)


# speedup vs baseline: 1.2463x; 1.0082x over previous
import functools

import jax
import jax.numpy as jnp
from jax import lax
from jax.experimental import pallas as pl
from jax.experimental.pallas import tpu as pltpu
from jax.experimental.pallas import tpu_sc as plsc

F32 = jnp.float32

_NC = 2
_NS = 16
_NW = _NC * _NS


def _elu(y):
    return jnp.where(y > 0.0, y, jnp.exp(jnp.minimum(y, 0.0)) - 1.0)


def _proj_body(nf_ref, w_ref, pr_ref, q_ref, s_ref):
    c = nf_ref.shape[1]
    t = jnp.dot(nf_ref[...], w_ref[...], preferred_element_type=F32)
    pr_ref[...] = t[:, : 2 * c]
    q_ref[...] = t[:, 2 * c : 3 * c]
    s_ref[...] = t[:, 3 * c : 4 * c]


def _proj(nf, wcat, bn):
    n, c = nf.shape
    grid = (n // bn,)
    return pl.pallas_call(
        _proj_body,
        grid=grid,
        in_specs=[
            pl.BlockSpec((bn, c), lambda i: (i, 0)),
            pl.BlockSpec((c, 4 * c), lambda i: (0, 0)),
        ],
        out_specs=[
            pl.BlockSpec((bn, 2 * c), lambda i: (i, 0)),
            pl.BlockSpec((bn, c), lambda i: (i, 0)),
            pl.BlockSpec((bn, c), lambda i: (i, 0)),
        ],
        out_shape=[
            jax.ShapeDtypeStruct((n, 2 * c), F32),
            jax.ShapeDtypeStruct((n, c), F32),
            jax.ShapeDtypeStruct((n, c), F32),
        ],
    )(nf, wcat)


def _gather_call(pr, q, src, dst, e, c, k):
    ept = e // _NW
    nchunk = ept // k
    tail = ept - nchunk * k
    mesh = plsc.VectorSubcoreMesh(core_axis_name="c", subcore_axis_name="s")

    @functools.partial(
        pl.kernel,
        out_type=[
            jax.ShapeDtypeStruct((e, 2 * c), F32),
            jax.ShapeDtypeStruct((e, c), F32),
        ],
        mesh=mesh,
        scratch_types=[
            pltpu.VMEM((k,), jnp.int32),
            pltpu.VMEM((k,), jnp.int32),
            pltpu.VMEM((k,), jnp.int32),
            pltpu.VMEM((k,), jnp.int32),
            pltpu.VMEM((k, 2 * c), F32),
            pltpu.VMEM((k, 2 * c), F32),
            pltpu.VMEM((k, c), F32),
            pltpu.VMEM((k, c), F32),
            pltpu.SemaphoreType.DMA,
            pltpu.SemaphoreType.DMA,
        ],
        compiler_params=pltpu.CompilerParams(use_tc_tiling_on_sc=False),
    )
    def kern(pr_h, q_h, src_h, dst_h, gpr_h, gq_h,
             sv0, sv1, dv0, dv1, pb0, pb1, qb0, qb1, sm0, sm1):
        wid = lax.axis_index("s") * _NC + lax.axis_index("c")
        base = wid * ept
        svs, dvs = (sv0, sv1), (dv0, dv1)
        pbs, qbs, sems = (pb0, pb1), (qb0, qb1), (sm0, sm1)

        def start_gather(ci, b):
            cb = base + ci * k
            pltpu.sync_copy(src_h.at[pl.ds(cb, k)], svs[b])
            pltpu.sync_copy(dst_h.at[pl.ds(cb, k)], dvs[b])
            pltpu.async_copy(pr_h.at[svs[b]], pbs[b], sems[b])
            pltpu.async_copy(q_h.at[dvs[b]], qbs[b], sems[b])

        def drain_gather(b):
            pltpu.make_async_copy(pr_h.at[svs[b]], pbs[b], sems[b]).wait()
            pltpu.make_async_copy(q_h.at[dvs[b]], qbs[b], sems[b]).wait()

        def write_out(ci, b):
            cb = base + ci * k
            pltpu.sync_copy(pbs[b], gpr_h.at[pl.ds(cb, k), :])
            pltpu.sync_copy(qbs[b], gq_h.at[pl.ds(cb, k), :])

        start_gather(0, 0)

        def chunk2(c2, carry):
            ci = c2 * 2
            start_gather(ci + 1, 1)
            drain_gather(0)
            write_out(ci, 0)
            nxt = jnp.minimum(ci + 2, nchunk - 1)
            start_gather(nxt, 0)
            drain_gather(1)
            write_out(ci + 1, 1)
            return carry

        lax.fori_loop(0, nchunk // 2, chunk2, 0)
        drain_gather(0)

        if tail:
            cb = base + nchunk * k
            pltpu.sync_copy(src_h.at[pl.ds(cb, tail)], sv0.at[pl.ds(0, tail)])
            pltpu.sync_copy(dst_h.at[pl.ds(cb, tail)], dv0.at[pl.ds(0, tail)])
            cp1 = pltpu.async_copy(pr_h.at[sv0.at[pl.ds(0, tail)]],
                                   pb0.at[pl.ds(0, tail), :], sm0)
            cp2 = pltpu.async_copy(q_h.at[dv0.at[pl.ds(0, tail)]],
                                   qb0.at[pl.ds(0, tail), :], sm0)
            cp1.wait()
            cp2.wait()
            pltpu.sync_copy(pb0.at[pl.ds(0, tail), :],
                            gpr_h.at[pl.ds(cb, tail), :])
            pltpu.sync_copy(qb0.at[pl.ds(0, tail), :],
                            gq_h.at[pl.ds(cb, tail), :])

    return kern(pr, q, src, dst)


def _edge1_body(ef_ref, gp_ref, gq_ref, w_ref, z1_ref, st_ref):
    z = jnp.dot(ef_ref[...], w_ref[...], preferred_element_type=F32)
    z = z + gp_ref[...] + gq_ref[...]
    z1_ref[...] = z

    @pl.when(pl.program_id(0) == 0)
    def _():
        st_ref[...] = jnp.zeros_like(st_ref)

    s = jnp.sum(z, axis=0, keepdims=True)
    s2 = jnp.sum(z * z, axis=0, keepdims=True)
    st_ref[...] += jnp.concatenate([s, s2], axis=0)


def _edge1(ef, gpr, gq, w1c, be):
    e, c = ef.shape
    grid = (e // be,)
    return pl.pallas_call(
        _edge1_body,
        grid=grid,
        in_specs=[
            pl.BlockSpec((be, c), lambda i: (i, 0)),
            pl.BlockSpec((be, c), lambda i: (i, 0)),
            pl.BlockSpec((be, c), lambda i: (i, 0)),
            pl.BlockSpec((c, c), lambda i: (0, 0)),
        ],
        out_specs=[
            pl.BlockSpec((be, c), lambda i: (i, 0)),
            pl.BlockSpec((2, c), lambda i: (0, 0)),
        ],
        out_shape=[
            jax.ShapeDtypeStruct((e, c), F32),
            jax.ShapeDtypeStruct((2, c), F32),
        ],
    )(ef, gpr, gq, w1c)


def _edge2_body(nsteps, e, z1_ref, gr_ref, w_ref, st1_ref, g1_ref, be1_ref,
                g2_ref, be2_ref, ex_ref, z2_ref, af2_ref, acc_ref):
    inv_e = 1.0 / e
    mu = st1_ref[0:1, :] * inv_e
    var = jnp.maximum(st1_ref[1:2, :] * inv_e - mu * mu, 0.0)
    a1 = g1_ref[...] * lax.rsqrt(var + 1e-5)
    c1 = be1_ref[...] - mu * a1
    ex = _elu(z1_ref[...] * a1 + c1)
    ex_ref[...] = ex
    z2 = jnp.dot(ex, w_ref[...], preferred_element_type=F32) + gr_ref[...]
    z2_ref[...] = z2

    @pl.when(pl.program_id(0) == 0)
    def _():
        acc_ref[...] = jnp.zeros_like(acc_ref)

    s = jnp.sum(z2, axis=0, keepdims=True)
    s2 = jnp.sum(z2 * z2, axis=0, keepdims=True)
    acc_ref[...] += jnp.concatenate([s, s2], axis=0)

    @pl.when(pl.program_id(0) == nsteps - 1)
    def _():
        mu2 = acc_ref[0:1, :] * inv_e
        var2 = jnp.maximum(acc_ref[1:2, :] * inv_e - mu2 * mu2, 0.0)
        a2 = g2_ref[...] * lax.rsqrt(var2 + 1e-5)
        c2 = be2_ref[...] - mu2 * a2
        af2_ref[...] = jnp.concatenate([a2, c2], axis=0)


def _edge2(z1, gpr, w2b, st1, g1, be1, g2, be2, be):
    e, c = z1.shape
    nsteps = e // be
    return pl.pallas_call(
        functools.partial(_edge2_body, nsteps, e),
        grid=(nsteps,),
        in_specs=[
            pl.BlockSpec((be, c), lambda i: (i, 0)),
            pl.BlockSpec((be, c), lambda i: (i, 1)),
            pl.BlockSpec((c, c), lambda i: (0, 0)),
            pl.BlockSpec((2, c), lambda i: (0, 0)),
            pl.BlockSpec((1, c), lambda i: (0, 0)),
            pl.BlockSpec((1, c), lambda i: (0, 0)),
            pl.BlockSpec((1, c), lambda i: (0, 0)),
            pl.BlockSpec((1, c), lambda i: (0, 0)),
        ],
        out_specs=[
            pl.BlockSpec((be, c), lambda i: (i, 0)),
            pl.BlockSpec((be, c), lambda i: (i, 0)),
            pl.BlockSpec((2, c), lambda i: (0, 0)),
        ],
        out_shape=[
            jax.ShapeDtypeStruct((e, c), F32),
            jax.ShapeDtypeStruct((e, c), F32),
            jax.ShapeDtypeStruct((2, c), F32),
        ],
        scratch_shapes=[pltpu.VMEM((2, c), F32)],
    )(z1, gpr, w2b, st1, g1, be1, g2, be2)


def _mpass_body(z2_ref, af2_ref, m_ref):
    be = z2_ref.shape[0]
    m = _elu(z2_ref[...] * af2_ref[0:1, :] + af2_ref[1:2, :])
    m_ref[...] = jnp.concatenate([m, jnp.ones((be, 16), F32)], axis=1)


def _mpass(z2, af2, be):
    e, c = z2.shape
    return pl.pallas_call(
        _mpass_body,
        grid=(e // be,),
        in_specs=[
            pl.BlockSpec((be, c), lambda i: (i, 0)),
            pl.BlockSpec((2, c), lambda i: (0, 0)),
        ],
        out_specs=pl.BlockSpec((be, c + 16), lambda i: (i, 0)),
        out_shape=jax.ShapeDtypeStruct((e, c + 16), F32),
    )(z2, af2)


def _scatter_call(m144, dst, n, e, c, k):
    ept = e // _NS
    nchunk = ept // k
    tail = ept - nchunk * k
    half = -(-n // _NC)
    rpt = (-(-(half + 1) // _NS) + 7) // 8 * 8
    nh_pad = rpt * _NS
    cw = c + 16
    ki = k // 128
    mesh = plsc.VectorSubcoreMesh(core_axis_name="c", subcore_axis_name="s")

    @functools.partial(
        pl.kernel,
        out_type=jax.ShapeDtypeStruct((_NC, nh_pad, cw), F32),
        mesh=mesh,
        scratch_types=[
            pltpu.VMEM((k, cw), F32),
            pltpu.VMEM((k, cw), F32),
            pltpu.VMEM((k,), jnp.int32),
            pltpu.VMEM((k,), jnp.int32),
            pltpu.VMEM((ki, 128), jnp.int32),
            pltpu.VMEM((rpt, cw), F32),
            pltpu.VMEM_SHARED((nh_pad, cw), F32),
            pltpu.SemaphoreType.DMA,
            pltpu.SemaphoreType.DMA,
        ],
        compiler_params=pltpu.CompilerParams(use_tc_tiling_on_sc=False),
    )
    def kern(m_h, dst_h, out_h, mb0, mb1, dv0, dv1, idxb, dbuf, acc, sm0, sm1):
        cid = lax.axis_index("c")
        sid = lax.axis_index("s")
        base = sid * ept
        r0 = sid * rpt
        off = cid * half
        mbufs, dvs, sems = (mb0, mb1), (dv0, dv1), (sm0, sm1)

        def zrow(i, carry):
            for j in range(cw // 16):
                dbuf[i, pl.ds(j * 16, 16)] = jnp.zeros((16,), F32)
            return carry

        lax.fori_loop(0, rpt, zrow, 0)
        pltpu.sync_copy(dbuf, acc.at[pl.ds(r0, rpt)])
        plsc.subcore_barrier()

        def start_load(ci, b):
            cb = base + ci * k
            pltpu.async_copy(m_h.at[pl.ds(cb, k), :], mbufs[b], sems[b])
            pltpu.async_copy(dst_h.at[pl.ds(cb, k)], dvs[b], sems[b])

        def drain_load(b):
            pltpu.make_async_copy(m_h.at[pl.ds(0, k), :], mbufs[b],
                                  sems[b]).wait()
            pltpu.make_async_copy(dst_h.at[pl.ds(0, k)], dvs[b],
                                  sems[b]).wait()

        def process(b):
            for t in range(k // 16):
                sl = pl.ds(t * 16, 16)
                d = dvs[b][sl] - off
                ok = (d >= 0) & (d < half)
                idxb[t // 8, pl.ds((t % 8) * 16, 16)] = jnp.where(ok, d, half)
            for t in range(0, k, 128):
                pltpu.sync_copy(mbufs[b].at[pl.ds(t, 128), :],
                                acc.at[idxb.at[t // 128]], add=True)

        start_load(0, 0)

        def chunk2(c2, carry):
            ci = c2 * 2
            start_load(ci + 1, 1)
            drain_load(0)
            process(0)
            nxt = jnp.minimum(ci + 2, nchunk - 1)
            start_load(nxt, 0)
            drain_load(1)
            process(1)
            return carry

        lax.fori_loop(0, nchunk // 2, chunk2, 0)
        drain_load(0)

        if tail:
            cb = base + nchunk * k
            nrow = tail
            pltpu.sync_copy(m_h.at[pl.ds(cb, nrow), :],
                            mb0.at[pl.ds(0, nrow), :])
            pltpu.sync_copy(dst_h.at[pl.ds(cb, nrow)], dv0.at[pl.ds(0, nrow)])
            prow = -(-nrow // 128) * 128
            for t in range(nrow // 16):
                sl = pl.ds(t * 16, 16)
                d = dv0[sl] - off
                ok = (d >= 0) & (d < half)
                idxb[t // 8, pl.ds((t % 8) * 16, 16)] = jnp.where(ok, d, half)
            for t in range(nrow // 16, prow // 16):
                idxb[t // 8, pl.ds((t % 8) * 16, 16)] = jnp.full(
                    (16,), half, jnp.int32)
            for t in range(0, prow, 128):
                pltpu.sync_copy(mb0.at[pl.ds(t, 128), :],
                                acc.at[idxb.at[t // 128]], add=True)
        plsc.subcore_barrier()

        pltpu.sync_copy(acc.at[pl.ds(r0, rpt)], dbuf)
        pltpu.sync_copy(dbuf, out_h.at[cid, pl.ds(r0, rpt), :])

    return kern(m144, dst)


def _node_body(m0_ref, c0_ref, s_ref, w_ref, g3_ref, be3_ref, ho_ref):
    n = m0_ref.shape[0]
    msum = m0_ref[...]
    cnt = c0_ref[...][:, 0:1]
    hm = msum / jnp.maximum(cnt, 1.0)
    z3 = s_ref[...] + jnp.dot(hm, w_ref[...], preferred_element_type=F32)
    mu = jnp.sum(z3, axis=0, keepdims=True) * (1.0 / n)
    var = jnp.maximum(jnp.sum(z3 * z3, axis=0, keepdims=True) * (1.0 / n)
                      - mu * mu, 0.0)
    a3 = g3_ref[...] * lax.rsqrt(var + 1e-5)
    c3 = be3_ref[...] - mu * a3
    ho_ref[...] = _elu(z3 * a3 + c3)


def _node(m0, c0, s, w3b, g3, be3):
    n, c = s.shape
    return pl.pallas_call(
        _node_body,
        in_specs=[pl.BlockSpec(x.shape, lambda: tuple(0 for _ in x.shape))
                  for x in (m0, c0, s, w3b, g3, be3)],
        out_specs=pl.BlockSpec((n, c), lambda: (0, 0)),
        out_shape=jax.ShapeDtypeStruct((n, c), F32),
    )(m0, c0, s, w3b, g3, be3)


def kernel(node_feats, edge_feats, edge_index, W1, b1, g1, be1,
           W2, b2, g2, be2, W3, b3, g3, be3):
    n, c = node_feats.shape
    e = edge_feats.shape[0]
    src = edge_index[0]
    dst = edge_index[1]

    wcat = jnp.concatenate(
        [W1[:c], W2[:c], W1[c : 2 * c], W3[:c]], axis=1)
    w1c = W1[2 * c : 3 * c]
    w2b = W2[c : 2 * c]
    w3b = W3[c : 2 * c]
    g1r, be1r, g2r, be2r, g3r, be3r = (
        x.reshape(1, c) for x in (g1, be1, g2, be2, g3, be3))

    pr, q, s = _proj(node_feats, wcat, bn=2000)
    gpr, gq = _gather_call(pr, q, src, dst, e, c, k=128)
    z1, st1 = _edge1(edge_feats, gpr, gq, w1c, be=8000)
    ex, z2, af2 = _edge2(z1, gpr, w2b, st1, g1r, be1r, g2r, be2r, be=8000)
    m144 = _mpass(z2, af2, be=8000)
    part = _scatter_call(m144, dst, n, e, c, k=128)
    half = -(-n // _NC)
    msum = jnp.concatenate([part[0, :half, :c], part[1, : n - half, :c]])
    cnt16 = jnp.concatenate(
        [part[0, :half, c : c + 16], part[1, : n - half, c : c + 16]])
    ho = _node(msum, cnt16, s, w3b, g3r, be3r)
    return (ho, ex)
